# baseline (speedup 1.0000x reference)
import functools

import jax
import jax.numpy as jnp
from jax.experimental import pallas as pl
from jax.experimental.pallas import tpu as pltpu

DIM = 64
CB = 1024
NQ = 8


NCHAIN = 4


def _rvq_kernel(z_ref, cbm2_ref, cn_ref, cb1_ref, cb2_ref, cb3_ref,
                q_ref, tok_ref, loss_ref):
    blk = z_ref.shape[0]
    sub = blk // NCHAIN
    iota = jax.lax.broadcasted_iota(jnp.int32, (sub, CB), 1)
    iota = iota.astype(jnp.float32)
    resid = [z_ref[h * sub:(h + 1) * sub, :] for h in range(NCHAIN)]
    qtot = [jnp.zeros((sub, DIM), jnp.float32) for _ in range(NCHAIN)]
    idx_cols = [[] for _ in range(NCHAIN)]
    loss_parts = []
    for s in range(NQ):
        dn_t = (((1,), (1,)), ((), ()))
        dn = (((1,), (0,)), ((), ()))
        loss_s = []
        for h in range(NCHAIN):
            rn = jnp.sum(resid[h] * resid[h], axis=1, keepdims=True)
            mm2 = jax.lax.dot_general(
                resid[h].astype(jnp.bfloat16), cbm2_ref[s], dn_t,
                preferred_element_type=jnp.float32)
            dist = (rn + mm2) + cn_ref[s][None, :]
            m = jnp.min(dist, axis=1, keepdims=True)
            idxf = jnp.min(jnp.where(dist == m, iota, float(CB)), axis=1,
                           keepdims=True)
            oh = (iota == idxf).astype(jnp.bfloat16)
            c1 = jax.lax.dot_general(oh, cb1_ref[s], dn,
                                     preferred_element_type=jnp.float32)
            c2 = jax.lax.dot_general(oh, cb2_ref[s], dn,
                                     preferred_element_type=jnp.float32)
            c3 = jax.lax.dot_general(oh, cb3_ref[s], dn,
                                     preferred_element_type=jnp.float32)
            codes = (c1 + c2) + c3
            d = resid[h] - codes
            loss_s.append(jnp.sum(d * d))
            qtot[h] = qtot[h] + codes
            resid[h] = d
            idx_cols[h].append(idxf.astype(jnp.int32))
        loss_parts.append(sum(loss_s))
    for h in range(NCHAIN):
        q_ref[h * sub:(h + 1) * sub, :] = qtot[h]
        tok_ref[h * sub:(h + 1) * sub, :] = jnp.concatenate(
            idx_cols[h], axis=1)
    loss_ref[...] = jnp.stack(loss_parts).reshape(1, 1, NQ)


@jax.jit
def kernel(z, codebooks):
    orig_shape = z.shape
    flat = z.reshape(-1, DIM)
    n = flat.shape[0]
    blk = 2048
    nblk = n // blk
    cn = jnp.sum(codebooks ** 2, axis=-1)
    def _trunc_bf16(x):
        u = jax.lax.bitcast_convert_type(x, jnp.uint32)
        return jax.lax.bitcast_convert_type(
            u & jnp.uint32(0xFFFF0000), jnp.float32)
    t1 = _trunc_bf16(codebooks)
    r1 = codebooks - t1
    t2 = _trunc_bf16(r1)
    cb1 = t1.astype(jnp.bfloat16)
    cb2 = t2.astype(jnp.bfloat16)
    cb3 = (r1 - t2).astype(jnp.bfloat16)
    cbm2 = (-2.0 * codebooks).astype(jnp.bfloat16)

    q, tok, loss = pl.pallas_call(
        _rvq_kernel,
        grid=(nblk,),
        in_specs=[
            pl.BlockSpec((blk, DIM), lambda i: (i, 0)),
            pl.BlockSpec((NQ, CB, DIM), lambda i: (0, 0, 0)),
            pl.BlockSpec((NQ, CB), lambda i: (0, 0)),
            pl.BlockSpec((NQ, CB, DIM), lambda i: (0, 0, 0)),
            pl.BlockSpec((NQ, CB, DIM), lambda i: (0, 0, 0)),
            pl.BlockSpec((NQ, CB, DIM), lambda i: (0, 0, 0)),
        ],
        out_specs=[
            pl.BlockSpec((blk, DIM), lambda i: (i, 0)),
            pl.BlockSpec((blk, NQ), lambda i: (i, 0)),
            pl.BlockSpec((1, 1, NQ), lambda i: (i, 0, 0)),
        ],
        out_shape=[
            jax.ShapeDtypeStruct((n, DIM), jnp.float32),
            jax.ShapeDtypeStruct((n, NQ), jnp.int32),
            jax.ShapeDtypeStruct((nblk, 1, NQ), jnp.float32),
        ],
        compiler_params=pltpu.CompilerParams(
            dimension_semantics=("parallel",)),
    )(flat, cbm2, cn, cb1, cb2, cb3)

    quantized = q.reshape(orig_shape)
    tokens = tok.reshape(orig_shape[:-1] + (NQ,))
    commit_loss = jnp.sum(loss) * (1.25 / (NQ * n * DIM))
    return quantized, tokens, commit_loss


# speedup vs baseline: 1.3964x; 1.3964x over previous
import functools

import jax
import jax.numpy as jnp
from jax.experimental import pallas as pl
from jax.experimental.pallas import tpu as pltpu

DIM = 64
CB = 1024
NQ = 8


NCHAIN = 2


def _rvq_kernel(z_ref, cbm2_ref, cn_ref, cbs_ref,
                q_ref, tok_ref, loss_ref):
    blk = z_ref.shape[0]
    sub = blk // NCHAIN
    iota = jax.lax.broadcasted_iota(jnp.int32, (sub, CB), 1)
    iota = iota.astype(jnp.float32)
    resid = [z_ref[h * sub:(h + 1) * sub, :] for h in range(NCHAIN)]
    qtot = [jnp.zeros((sub, DIM), jnp.float32) for _ in range(NCHAIN)]
    idx_cols = [[] for _ in range(NCHAIN)]
    loss_parts = []
    for s in range(NQ):
        dn_t = (((1,), (1,)), ((), ()))
        dn = (((1,), (0,)), ((), ()))
        loss_s = []
        for h in range(NCHAIN):
            rn = jnp.sum(resid[h] * resid[h], axis=1, keepdims=True)
            mm2 = jax.lax.dot_general(
                resid[h].astype(jnp.bfloat16), cbm2_ref[s], dn_t,
                preferred_element_type=jnp.float32)
            dist = (rn + mm2) + cn_ref[s][None, :]
            m = jnp.min(dist, axis=1, keepdims=True)
            idxf = jnp.min(jnp.where(dist == m, iota, float(CB)), axis=1,
                           keepdims=True)
            oh = (iota == idxf).astype(jnp.bfloat16)
            gg = jax.lax.dot_general(oh, cbs_ref[s], dn,
                                     preferred_element_type=jnp.float32)
            codes = (gg[:, 0:DIM] + gg[:, 128:128 + DIM]) \
                + gg[:, 256:256 + DIM]
            d = resid[h] - codes
            loss_s.append(jnp.sum(d * d))
            qtot[h] = qtot[h] + codes
            resid[h] = d
            idx_cols[h].append(idxf.astype(jnp.int32))
        loss_parts.append(sum(loss_s))
    for h in range(NCHAIN):
        q_ref[h * sub:(h + 1) * sub, :] = qtot[h]
        tok_ref[h * sub:(h + 1) * sub, :] = jnp.concatenate(
            idx_cols[h], axis=1)
    loss_ref[...] = jnp.stack(loss_parts).reshape(1, 1, NQ)


@jax.jit
def kernel(z, codebooks):
    orig_shape = z.shape
    flat = z.reshape(-1, DIM)
    n = flat.shape[0]
    blk = 1024
    nblk = n // blk
    cn = jnp.sum(codebooks ** 2, axis=-1)
    def _trunc_bf16(x):
        u = jax.lax.bitcast_convert_type(x, jnp.uint32)
        return jax.lax.bitcast_convert_type(
            u & jnp.uint32(0xFFFF0000), jnp.float32)
    t1 = _trunc_bf16(codebooks)
    r1 = codebooks - t1
    t2 = _trunc_bf16(r1)
    pad = jnp.zeros((NQ, CB, 128 - DIM), jnp.float32)
    cbs = jnp.concatenate(
        [t1, pad, t2, pad, r1 - t2, pad], axis=-1).astype(jnp.bfloat16)
    cbm2 = (-2.0 * codebooks).astype(jnp.bfloat16)

    q, tok, loss = pl.pallas_call(
        _rvq_kernel,
        grid=(nblk,),
        in_specs=[
            pl.BlockSpec((blk, DIM), lambda i: (i, 0)),
            pl.BlockSpec((NQ, CB, DIM), lambda i: (0, 0, 0)),
            pl.BlockSpec((NQ, CB), lambda i: (0, 0)),
            pl.BlockSpec((NQ, CB, 384), lambda i: (0, 0, 0)),
        ],
        out_specs=[
            pl.BlockSpec((blk, DIM), lambda i: (i, 0)),
            pl.BlockSpec((blk, NQ), lambda i: (i, 0)),
            pl.BlockSpec((1, 1, NQ), lambda i: (i, 0, 0)),
        ],
        out_shape=[
            jax.ShapeDtypeStruct((n, DIM), jnp.float32),
            jax.ShapeDtypeStruct((n, NQ), jnp.int32),
            jax.ShapeDtypeStruct((nblk, 1, NQ), jnp.float32),
        ],
        compiler_params=pltpu.CompilerParams(
            dimension_semantics=("parallel",)),
    )(flat, cbm2, cn, cbs)

    quantized = q.reshape(orig_shape)
    tokens = tok.reshape(orig_shape[:-1] + (NQ,))
    commit_loss = jnp.sum(loss) * (1.25 / (NQ * n * DIM))
    return quantized, tokens, commit_loss
